# Initial kernel scaffold; baseline (speedup 1.0000x reference)
import jax
import jax.numpy as jnp
from jax.experimental import pallas as pl


def kernel(x, cat_vec, params):
    raise NotImplementedError("write your pallas kernel here")


# speedup vs baseline: 10.8672x; 10.8672x over previous
import functools

import jax
import jax.numpy as jnp
import numpy as np
from jax import lax
from jax.experimental import pallas as pl
from jax.experimental.pallas import tpu as pltpu
from jax.experimental.pallas import tpu_sc as plsc

_PD = 16


def _pad_pos(p):
    return jnp.pad(p, ((0, 0), (0, 0), (0, _PD - p.shape[-1])))


def _pad_w(w, rows):
    return jnp.pad(w, ((0, rows - w.shape[0]), (0, 0)))


def _row(b):
    return b.reshape(1, -1)


def _sc_gather(table, idx):
    R, D = table.shape
    rows = idx.shape[0]
    NW = 32
    bpw = rows // NW
    ch = bpw
    while ch > 128 or ch * D * 4 > 196608:
        ch //= 2
    nch = bpw // ch
    mesh = plsc.VectorSubcoreMesh(core_axis_name="c", subcore_axis_name="s")

    @functools.partial(
        pl.kernel,
        out_type=jax.ShapeDtypeStruct((rows, D), jnp.float32),
        mesh=mesh,
        compiler_params=pltpu.CompilerParams(use_tc_tiling_on_sc=False),
        scratch_types=[
            pltpu.VMEM((ch,), jnp.int32),
            pltpu.VMEM((ch, D), jnp.float32),
            pltpu.SemaphoreType.DMA,
        ],
    )
    def gk(tbl, ix, out, ixv, rbuf, sem):
        wid = lax.axis_index("s") * 2 + lax.axis_index("c")

        def step(c, carry):
            base = wid * bpw + c * ch
            pltpu.sync_copy(ix.at[pl.ds(base, ch)], ixv)
            pltpu.async_copy(tbl.at[ixv], rbuf, sem).wait()
            pltpu.sync_copy(rbuf, out.at[pl.ds(base, ch)])
            return carry

        lax.fori_loop(0, nch, step, 0)

    return gk(table, idx)


def _gather_rows(table_bnd, idx_bmk):
    Bq, N, D = table_bnd.shape
    _, M, K = idx_bmk.shape
    flat = (
        idx_bmk + (jnp.arange(Bq, dtype=jnp.int32) * N)[:, None, None]
    ).reshape(-1)
    rows = flat.shape[0]
    pad = (-rows) % 256
    if pad:
        flat = jnp.concatenate([flat, jnp.zeros((pad,), jnp.int32)])
    out = _sc_gather(table_bnd.reshape(Bq * N, D), flat)
    if pad:
        out = out[:rows]
    return out.reshape(Bq, M, K, D)


def _knn(q_pos, s_pos, k):
    B, M, _ = q_pos.shape
    N = s_pos.shape[1]
    BM = 256 if (M > 256 and M % 256 == 0) else M

    def body(q_ref, s_ref, idx_ref, val_ref):
        q = q_ref[0]
        s = s_ref[0]
        d = (
            jnp.sum(q * q, axis=-1, keepdims=True)
            + jnp.sum(s * s, axis=-1)[None, :]
            - 2.0
            * lax.dot_general(
                q, s, (((1,), (1,)), ((), ())),
                preferred_element_type=jnp.float32,
            )
        )
        iota = lax.broadcasted_iota(jnp.int32, (BM, N), 1)
        idxs, vals = [], []
        for _ in range(k):
            m = jnp.min(d, axis=-1, keepdims=True)
            j = jnp.min(
                jnp.where(d == m, iota, N), axis=-1, keepdims=True
            )
            idxs.append(j)
            vals.append(m)
            d = jnp.where(iota == j, jnp.float32(np.inf), d)
        idx_ref[0] = jnp.concatenate(idxs, axis=-1)
        val_ref[0] = jnp.concatenate(vals, axis=-1)

    return pl.pallas_call(
        body,
        grid=(B, M // BM),
        in_specs=[
            pl.BlockSpec((1, BM, _PD), lambda b, m: (b, m, 0)),
            pl.BlockSpec((1, N, _PD), lambda b, m: (b, 0, 0)),
        ],
        out_specs=[
            pl.BlockSpec((1, BM, k), lambda b, m: (b, m, 0)),
            pl.BlockSpec((1, BM, k), lambda b, m: (b, m, 0)),
        ],
        out_shape=[
            jax.ShapeDtypeStruct((B, M, k), jnp.int32),
            jax.ShapeDtypeStruct((B, M, k), jnp.float32),
        ],
    )(q_pos, s_pos)


def _fps(xyz_pad, npoint):
    B, N, _ = xyz_pad.shape
    xyz_t = jnp.swapaxes(xyz_pad, 1, 2)

    def body(x_ref, out_ref):
        xyzt = x_ref[...]
        iota_n = lax.broadcasted_iota(jnp.int32, (B, N), 1)
        iota_p = lax.broadcasted_iota(jnp.int32, (B, npoint), 1)

        def step(i, carry):
            dist, far, nxyzt = carry
            m1 = jnp.where(iota_n == far, 1.0, 0.0)
            cent = jnp.sum(xyzt * m1[:, None, :], axis=2)
            m2 = jnp.where(iota_p == i, 1.0, 0.0)
            nxyzt = nxyzt + m2[:, None, :] * cent[:, :, None]
            d = jnp.sum((xyzt - cent[:, :, None]) ** 2, axis=1)
            dist = jnp.minimum(dist, d)
            mx = jnp.max(dist, axis=-1, keepdims=True)
            far = jnp.min(
                jnp.where(dist == mx, iota_n, N), axis=-1, keepdims=True
            )
            return dist, far, nxyzt

        init = (
            jnp.full((B, N), 1e10, jnp.float32),
            jnp.zeros((B, 1), jnp.int32),
            jnp.zeros((B, _PD, npoint), jnp.float32),
        )
        _, _, nxyzt = lax.fori_loop(0, npoint, step, init)
        out_ref[...] = nxyzt

    out_t = pl.pallas_call(
        body,
        out_shape=jax.ShapeDtypeStruct((B, _PD, npoint), jnp.float32),
    )(xyz_t)
    return jnp.swapaxes(out_t, 1, 2)


def _mlp(x, layers):
    B, N, _ = x.shape
    flats = []
    for W, b, _r in layers:
        flats.append(W)
        flats.append(_row(b))
    dout = layers[-1][0].shape[1]

    def body(x_ref, *refs):
        out_ref = refs[-1]
        h = x_ref[0]
        for li, (_W, _b, r) in enumerate(layers):
            h = (
                jnp.dot(
                    h, refs[2 * li][...],
                    preferred_element_type=jnp.float32,
                )
                + refs[2 * li + 1][...]
            )
            if r:
                h = jnp.maximum(h, 0.0)
        out_ref[0] = h

    in_specs = [pl.BlockSpec((1, N, x.shape[2]), lambda b: (b, 0, 0))]
    for f in flats:
        in_specs.append(
            pl.BlockSpec(f.shape, lambda b: (0,) * f.ndim)
        )
    return pl.pallas_call(
        body,
        grid=(B,),
        in_specs=in_specs,
        out_specs=pl.BlockSpec((1, N, dout), lambda b: (b, 0, 0)),
        out_shape=jax.ShapeDtypeStruct((B, N, dout), jnp.float32),
    )(x, *flats)


def _ptb_pre(x, pos_pad, p):
    B, N, d = x.shape
    td = p["fc1"]["W"].shape[1]
    w1, b1 = p["fc1"]["W"], _row(p["fc1"]["b"])
    wq, wk, wv = p["w_qs"]["W"], p["w_ks"]["W"], p["w_vs"]["W"]

    def body(x_ref, pos_ref, w1r, b1r, wqr, wkr, wvr, q_ref, t_ref):
        h = (
            jnp.dot(x_ref[0], w1r[...], preferred_element_type=jnp.float32)
            + b1r[...]
        )
        q_ref[0] = jnp.dot(h, wqr[...], preferred_element_type=jnp.float32)
        kk = jnp.dot(h, wkr[...], preferred_element_type=jnp.float32)
        vv = jnp.dot(h, wvr[...], preferred_element_type=jnp.float32)
        t_ref[0] = jnp.concatenate([kk, vv, pos_ref[0]], axis=-1)

    D2 = 2 * td + _PD
    in_specs = [
        pl.BlockSpec((1, N, d), lambda b: (b, 0, 0)),
        pl.BlockSpec((1, N, _PD), lambda b: (b, 0, 0)),
    ] + [
        pl.BlockSpec(f.shape, lambda b: (0,) * f.ndim)
        for f in (w1, b1, wq, wk, wv)
    ]
    return pl.pallas_call(
        body,
        grid=(B,),
        in_specs=in_specs,
        out_specs=[
            pl.BlockSpec((1, N, td), lambda b: (b, 0, 0)),
            pl.BlockSpec((1, N, D2), lambda b: (b, 0, 0)),
        ],
        out_shape=[
            jax.ShapeDtypeStruct((B, N, td), jnp.float32),
            jax.ShapeDtypeStruct((B, N, D2), jnp.float32),
        ],
    )(x, pos_pad, w1, b1, wq, wk, wv)


def _ptb_post(q, gath, pos_pad, x, p):
    B, N, td = q.shape
    K = gath.shape[2]
    d = x.shape[2]
    D2 = gath.shape[3]
    BN = 512 if (N > 512 and N % 512 == 0) else N
    wd0 = _pad_w(p["fc_delta"][0]["W"], _PD)
    bd0 = _row(p["fc_delta"][0]["b"])
    wd1, bd1 = p["fc_delta"][1]["W"], _row(p["fc_delta"][1]["b"])
    wg0, bg0 = p["fc_gamma"][0]["W"], _row(p["fc_gamma"][0]["b"])
    wg1, bg1 = p["fc_gamma"][1]["W"], _row(p["fc_gamma"][1]["b"])
    w2, b2 = p["fc2"]["W"], _row(p["fc2"]["b"])
    ws = (wd0, bd0, wd1, bd1, wg0, bg0, wg1, bg1, w2, b2)
    scale = 1.0 / np.sqrt(td).astype(np.float32)

    def body(q_ref, g_ref, pos_ref, x_ref, *refs):
        (wd0r, bd0r, wd1r, bd1r, wg0r, bg0r, wg1r, bg1r, w2r, b2r,
         out_ref) = refs
        qq = q_ref[0]
        g = g_ref[0]
        kg = g[..., :td]
        vg = g[..., td : 2 * td]
        kp = g[..., 2 * td :]
        delta = (pos_ref[0][:, None, :] - kp).reshape(BN * K, _PD)
        pe = jnp.maximum(
            jnp.dot(delta, wd0r[...], preferred_element_type=jnp.float32)
            + bd0r[...],
            0.0,
        )
        pe = (
            jnp.dot(pe, wd1r[...], preferred_element_type=jnp.float32)
            + bd1r[...]
        ).reshape(BN, K, td)
        t = (qq[:, None, :] - kg + pe).reshape(BN * K, td)
        a = jnp.maximum(
            jnp.dot(t, wg0r[...], preferred_element_type=jnp.float32)
            + bg0r[...],
            0.0,
        )
        a = (
            jnp.dot(a, wg1r[...], preferred_element_type=jnp.float32)
            + bg1r[...]
        ).reshape(BN, K, td) * scale
        m = jnp.max(a, axis=1, keepdims=True)
        e = jnp.exp(a - m)
        attn = e / jnp.sum(e, axis=1, keepdims=True)
        res = jnp.sum(attn * (vg + pe), axis=1)
        out_ref[0] = (
            jnp.dot(res, w2r[...], preferred_element_type=jnp.float32)
            + b2r[...]
            + x_ref[0]
        )

    in_specs = [
        pl.BlockSpec((1, BN, td), lambda b, n: (b, n, 0)),
        pl.BlockSpec((1, BN, K, D2), lambda b, n: (b, n, 0, 0)),
        pl.BlockSpec((1, BN, _PD), lambda b, n: (b, n, 0)),
        pl.BlockSpec((1, BN, d), lambda b, n: (b, n, 0)),
    ] + [
        pl.BlockSpec(f.shape, lambda b, n: (0,) * f.ndim) for f in ws
    ]
    return pl.pallas_call(
        body,
        grid=(B, N // BN),
        in_specs=in_specs,
        out_specs=pl.BlockSpec((1, BN, d), lambda b, n: (b, n, 0)),
        out_shape=jax.ShapeDtypeStruct((B, N, d), jnp.float32),
    )(q, gath, pos_pad, x, *ws)


def _ptb(x, pos_pad, knn_idx, p):
    q, table = _ptb_pre(x, pos_pad, p)
    gath = _gather_rows(table, knn_idx)
    return _ptb_post(q, gath, pos_pad, x, p)


def _td_post(gath, new_xyz_pad, p, C):
    B, M, K, Dg = gath.shape
    w1 = _pad_w(p["mlp1"]["W"], C + _PD)
    b1 = _row(p["mlp1"]["b"])
    w2, b2 = p["mlp2"]["W"], _row(p["mlp2"]["b"])
    bh = w2.shape[1]

    def body(g_ref, nx_ref, w1r, b1r, w2r, b2r, out_ref):
        g = g_ref[0]
        norm = g[..., C:] - nx_ref[0][:, None, :]
        feat = jnp.concatenate([g[..., :C], norm], axis=-1).reshape(
            M * K, C + _PD
        )
        h = jnp.maximum(
            jnp.dot(feat, w1r[...], preferred_element_type=jnp.float32)
            + b1r[...],
            0.0,
        )
        h = jnp.maximum(
            jnp.dot(h, w2r[...], preferred_element_type=jnp.float32)
            + b2r[...],
            0.0,
        )
        out_ref[0] = jnp.max(h.reshape(M, K, bh), axis=1)

    in_specs = [
        pl.BlockSpec((1, M, K, Dg), lambda b: (b, 0, 0, 0)),
        pl.BlockSpec((1, M, _PD), lambda b: (b, 0, 0)),
    ] + [
        pl.BlockSpec(f.shape, lambda b: (0,) * f.ndim)
        for f in (w1, b1, w2, b2)
    ]
    return pl.pallas_call(
        body,
        grid=(B,),
        in_specs=in_specs,
        out_specs=pl.BlockSpec((1, M, bh), lambda b: (b, 0, 0)),
        out_shape=jax.ShapeDtypeStruct((B, M, bh), jnp.float32),
    )(gath, new_xyz_pad, w1, b1, w2, b2)


def _tu_post(gath_flat, dvals, feats2):
    B, M2, d = feats2.shape

    def body(g_ref, dv_ref, f2_ref, out_ref):
        g = g_ref[0].reshape(M2, 3, d)
        d3 = jnp.maximum(dv_ref[0], 0.0)
        r = 1.0 / (d3 + 1e-8)
        w = r / jnp.sum(r, axis=-1, keepdims=True)
        out_ref[0] = jnp.sum(g * w[:, :, None], axis=1) + f2_ref[0]

    in_specs = [
        pl.BlockSpec((1, M2 * 3, d), lambda b: (b, 0, 0)),
        pl.BlockSpec((1, M2, 3), lambda b: (b, 0, 0)),
        pl.BlockSpec((1, M2, d), lambda b: (b, 0, 0)),
    ]
    return pl.pallas_call(
        body,
        grid=(B,),
        in_specs=in_specs,
        out_specs=pl.BlockSpec((1, M2, d), lambda b: (b, 0, 0)),
        out_shape=jax.ShapeDtypeStruct((B, M2, d), jnp.float32),
    )(gath_flat, dvals, feats2)


_N_NEI = 16
_N_BLOCKS = 4
_DSR = 4


def kernel(x, cat_vec, params):
    B, NPTS, _ = x.shape
    pos = _pad_pos(x)

    def lyr(p, r):
        return (p["W"], p["b"], r)

    h = _mlp(x, [lyr(params["bb_fc"][0], True), lyr(params["bb_fc"][1], False)])

    knn_cache = {}

    def knn_self(pos_pad):
        key = id(pos_pad)
        if key not in knn_cache:
            n = pos_pad.shape[1]
            k = min(_N_NEI, n)
            knn_cache[key] = _knn(pos_pad, pos_pad, k)[0]
        return knn_cache[key]

    h = _ptb(h, pos, knn_self(pos), params["bb_ptb"])
    hidden = [(pos, h)]
    for i in range(_N_BLOCKS):
        npoint = NPTS // _DSR ** (i + 1)
        new_pos = _fps(pos, npoint)
        k = min(_N_NEI, pos.shape[1])
        idx, _ = _knn(new_pos, pos, k)
        C = h.shape[2]
        table = jnp.concatenate([h, pos], axis=-1)
        gath = _gather_rows(table, idx)
        h = _td_post(gath, new_pos, params["td"][i], C)
        pos = new_pos
        h = _ptb(h, pos, knn_self(pos), params["bb_tf"][i])
        hidden.append((pos, h))

    h = _mlp(
        h,
        [
            lyr(params["seg_fc"][0], True),
            lyr(params["seg_fc"][1], True),
            lyr(params["seg_fc"][2], False),
        ],
    )
    h = _ptb(h, pos, knn_self(pos), params["seg_ptb"])

    for i in range(_N_BLOCKS):
        pos2, pts2 = hidden[-i - 2]
        tu = params["tu"][i]
        feats1 = _mlp(h, [lyr(tu["fc1"], True)])
        feats2 = _mlp(pts2, [lyr(tu["fc2"], True)])
        idx3, dv3 = _knn(pos2, pos, 3)
        gath = _gather_rows(feats1, idx3)
        d = feats1.shape[2]
        M2 = pos2.shape[1]
        h = _tu_post(gath.reshape(B, M2 * 3, d), dv3, feats2)
        pos = pos2
        h = _ptb(h, pos, knn_self(pos), params["seg_tf"][i])

    o = jnp.concatenate([h, cat_vec], axis=-1)
    o = _mlp(
        o,
        [
            lyr(params["out"][0], True),
            lyr(params["out"][1], True),
            lyr(params["out"][2], False),
        ],
    )
    return o


# speedup vs baseline: 12.1644x; 1.1194x over previous
import functools

import jax
import jax.numpy as jnp
import numpy as np
from jax import lax
from jax.experimental import pallas as pl
from jax.experimental.pallas import tpu as pltpu
from jax.experimental.pallas import tpu_sc as plsc

_PD = 16


def _pad_pos(p):
    return jnp.pad(p, ((0, 0), (0, 0), (0, _PD - p.shape[-1])))


def _pad_w(w, rows):
    return jnp.pad(w, ((0, rows - w.shape[0]), (0, 0)))


def _row(b):
    return b.reshape(1, -1)


def _sc_gather(table, idx):
    R, D = table.shape
    rows = idx.shape[0]
    NW = 32
    bpw = rows // NW
    ch = bpw
    while ch > 128 or ch * D * 4 > 196608:
        ch //= 2
    nch = bpw // ch
    mesh = plsc.VectorSubcoreMesh(core_axis_name="c", subcore_axis_name="s")

    @functools.partial(
        pl.kernel,
        out_type=jax.ShapeDtypeStruct((rows, D), jnp.float32),
        mesh=mesh,
        compiler_params=pltpu.CompilerParams(use_tc_tiling_on_sc=False),
        scratch_types=[
            pltpu.VMEM((bpw,), jnp.int32),
            pltpu.VMEM((ch, D), jnp.float32),
            pltpu.VMEM((ch, D), jnp.float32),
            pltpu.SemaphoreType.DMA,
            pltpu.SemaphoreType.DMA,
            pltpu.SemaphoreType.DMA,
            pltpu.SemaphoreType.DMA,
        ],
    )
    def gk(tbl, ix, out, ixv, rb0, rb1, sg0, sg1, ss0, ss1):
        wid = lax.axis_index("s") * 2 + lax.axis_index("c")
        base = wid * bpw
        rb = (rb0, rb1)
        sg = (sg0, sg1)
        ss = (ss0, ss1)
        pltpu.sync_copy(ix.at[pl.ds(base, bpw)], ixv)
        h_g = [None, None]
        h_s = [None, None]
        h_g[0] = pltpu.async_copy(
            tbl.at[ixv.at[pl.ds(0, ch)]], rb[0], sg[0]
        )
        for c in range(nch):
            p = c & 1
            if c + 1 < nch:
                if h_s[1 - p] is not None:
                    h_s[1 - p].wait()
                h_g[1 - p] = pltpu.async_copy(
                    tbl.at[ixv.at[pl.ds((c + 1) * ch, ch)]],
                    rb[1 - p],
                    sg[1 - p],
                )
            h_g[p].wait()
            h_s[p] = pltpu.async_copy(
                rb[p], out.at[pl.ds(base + c * ch, ch)], ss[p]
            )
        for p in (0, 1):
            if h_s[p] is not None:
                h_s[p].wait()

    return gk(table, idx)


def _gather_rows(table_bnd, idx_bmk):
    Bq, N, D = table_bnd.shape
    _, M, K = idx_bmk.shape
    flat = (
        idx_bmk + (jnp.arange(Bq, dtype=jnp.int32) * N)[:, None, None]
    ).reshape(-1)
    rows = flat.shape[0]
    pad = (-rows) % 256
    if pad:
        flat = jnp.concatenate([flat, jnp.zeros((pad,), jnp.int32)])
    out = _sc_gather(table_bnd.reshape(Bq * N, D), flat)
    if pad:
        out = out[:rows]
    return out.reshape(Bq, M, K, D)


def _knn(q_pos, s_pos, k, want_vals=False):
    B, M, _ = q_pos.shape
    N = s_pos.shape[1]
    BM = 256 if (M > 256 and M % 256 == 0) else M

    def body(q_ref, s_ref, idx_ref, *val_refs):
        q = q_ref[0]
        s = s_ref[0]
        d = (
            jnp.sum(q * q, axis=-1, keepdims=True)
            + jnp.sum(s * s, axis=-1)[None, :]
            - 2.0
            * lax.dot_general(
                q, s, (((1,), (1,)), ((), ())),
                preferred_element_type=jnp.float32,
            )
        )
        iota = lax.broadcasted_iota(jnp.int32, (BM, N), 1)
        idxs, vals = [], []
        for _ in range(k):
            if want_vals:
                m = jnp.min(d, axis=-1, keepdims=True)
                j = jnp.min(
                    jnp.where(d == m, iota, N), axis=-1, keepdims=True
                )
                vals.append(m)
            else:
                j = jnp.argmin(d, axis=-1).astype(jnp.int32)[:, None]
            idxs.append(j)
            d = jnp.where(iota == j, jnp.float32(np.inf), d)
        idx_ref[0] = jnp.concatenate(idxs, axis=-1)
        if want_vals:
            val_refs[0][0] = jnp.concatenate(vals, axis=-1)

    out_specs = [pl.BlockSpec((1, BM, k), lambda b, m: (b, m, 0))]
    out_shape = [jax.ShapeDtypeStruct((B, M, k), jnp.int32)]
    if want_vals:
        out_specs.append(pl.BlockSpec((1, BM, k), lambda b, m: (b, m, 0)))
        out_shape.append(jax.ShapeDtypeStruct((B, M, k), jnp.float32))
    res = pl.pallas_call(
        body,
        grid=(B, M // BM),
        in_specs=[
            pl.BlockSpec((1, BM, _PD), lambda b, m: (b, m, 0)),
            pl.BlockSpec((1, N, _PD), lambda b, m: (b, 0, 0)),
        ],
        out_specs=out_specs,
        out_shape=out_shape,
    )(q_pos, s_pos)
    return res if want_vals else (res[0], None)


def _fps(xyz_pad, npoint):
    B, N, _ = xyz_pad.shape
    xyz_t = jnp.swapaxes(xyz_pad, 1, 2)

    def body(x_ref, out_ref):
        xyzt = x_ref[...]
        iota_n = lax.broadcasted_iota(jnp.int32, (B, N), 1)
        iota_p = lax.broadcasted_iota(jnp.int32, (B, npoint), 1)

        def step(i, carry):
            dist, far, nxyzt = carry
            m1 = jnp.where(iota_n == far, 1.0, 0.0)
            cent = jnp.sum(xyzt * m1[:, None, :], axis=2)
            m2 = jnp.where(iota_p == i, 1.0, 0.0)
            nxyzt = nxyzt + m2[:, None, :] * cent[:, :, None]
            d = jnp.sum((xyzt - cent[:, :, None]) ** 2, axis=1)
            dist = jnp.minimum(dist, d)
            mx = jnp.max(dist, axis=-1, keepdims=True)
            far = jnp.min(
                jnp.where(dist == mx, iota_n, N), axis=-1, keepdims=True
            )
            return dist, far, nxyzt

        init = (
            jnp.full((B, N), 1e10, jnp.float32),
            jnp.zeros((B, 1), jnp.int32),
            jnp.zeros((B, _PD, npoint), jnp.float32),
        )
        _, _, nxyzt = lax.fori_loop(0, npoint, step, init)
        out_ref[...] = nxyzt

    out_t = pl.pallas_call(
        body,
        out_shape=jax.ShapeDtypeStruct((B, _PD, npoint), jnp.float32),
    )(xyz_t)
    return jnp.swapaxes(out_t, 1, 2)


def _mlp(x, layers):
    B, N, _ = x.shape
    flats = []
    for W, b, _r in layers:
        flats.append(W)
        flats.append(_row(b))
    dout = layers[-1][0].shape[1]

    def body(x_ref, *refs):
        out_ref = refs[-1]
        h = x_ref[0]
        for li, (_W, _b, r) in enumerate(layers):
            h = (
                jnp.dot(
                    h, refs[2 * li][...],
                    preferred_element_type=jnp.float32,
                )
                + refs[2 * li + 1][...]
            )
            if r:
                h = jnp.maximum(h, 0.0)
        out_ref[0] = h

    in_specs = [pl.BlockSpec((1, N, x.shape[2]), lambda b: (b, 0, 0))]
    for f in flats:
        in_specs.append(
            pl.BlockSpec(f.shape, lambda b: (0,) * f.ndim)
        )
    return pl.pallas_call(
        body,
        grid=(B,),
        in_specs=in_specs,
        out_specs=pl.BlockSpec((1, N, dout), lambda b: (b, 0, 0)),
        out_shape=jax.ShapeDtypeStruct((B, N, dout), jnp.float32),
    )(x, *flats)


def _ptb_pre(x, pos_pad, p):
    B, N, d = x.shape
    td = p["fc1"]["W"].shape[1]
    w1, b1 = p["fc1"]["W"], _row(p["fc1"]["b"])
    wq, wk, wv = p["w_qs"]["W"], p["w_ks"]["W"], p["w_vs"]["W"]

    def body(x_ref, pos_ref, w1r, b1r, wqr, wkr, wvr, q_ref, t_ref):
        h = (
            jnp.dot(x_ref[0], w1r[...], preferred_element_type=jnp.float32)
            + b1r[...]
        )
        q_ref[0] = jnp.dot(h, wqr[...], preferred_element_type=jnp.float32)
        kk = jnp.dot(h, wkr[...], preferred_element_type=jnp.float32)
        vv = jnp.dot(h, wvr[...], preferred_element_type=jnp.float32)
        t_ref[0] = jnp.concatenate([kk, vv, pos_ref[0]], axis=-1)

    D2 = 2 * td + _PD
    in_specs = [
        pl.BlockSpec((1, N, d), lambda b: (b, 0, 0)),
        pl.BlockSpec((1, N, _PD), lambda b: (b, 0, 0)),
    ] + [
        pl.BlockSpec(f.shape, lambda b: (0,) * f.ndim)
        for f in (w1, b1, wq, wk, wv)
    ]
    return pl.pallas_call(
        body,
        grid=(B,),
        in_specs=in_specs,
        out_specs=[
            pl.BlockSpec((1, N, td), lambda b: (b, 0, 0)),
            pl.BlockSpec((1, N, D2), lambda b: (b, 0, 0)),
        ],
        out_shape=[
            jax.ShapeDtypeStruct((B, N, td), jnp.float32),
            jax.ShapeDtypeStruct((B, N, D2), jnp.float32),
        ],
    )(x, pos_pad, w1, b1, wq, wk, wv)


def _ptb_post(q, gath, pos_pad, x, p):
    B, N, td = q.shape
    K = gath.shape[2]
    d = x.shape[2]
    D2 = gath.shape[3]
    BN = 512 if (N > 512 and N % 512 == 0) else N
    wd0 = _pad_w(p["fc_delta"][0]["W"], _PD)
    bd0 = _row(p["fc_delta"][0]["b"])
    wd1, bd1 = p["fc_delta"][1]["W"], _row(p["fc_delta"][1]["b"])
    wg0, bg0 = p["fc_gamma"][0]["W"], _row(p["fc_gamma"][0]["b"])
    wg1, bg1 = p["fc_gamma"][1]["W"], _row(p["fc_gamma"][1]["b"])
    w2, b2 = p["fc2"]["W"], _row(p["fc2"]["b"])
    ws = (wd0, bd0, wd1, bd1, wg0, bg0, wg1, bg1, w2, b2)
    scale = 1.0 / np.sqrt(td).astype(np.float32)

    def body(q_ref, g_ref, pos_ref, x_ref, *refs):
        (wd0r, bd0r, wd1r, bd1r, wg0r, bg0r, wg1r, bg1r, w2r, b2r,
         out_ref) = refs
        qq = q_ref[0]
        g = g_ref[0]
        kg = g[..., :td]
        vg = g[..., td : 2 * td]
        kp = g[..., 2 * td :]
        delta = (pos_ref[0][:, None, :] - kp).reshape(BN * K, _PD)
        pe = jnp.maximum(
            jnp.dot(delta, wd0r[...], preferred_element_type=jnp.float32)
            + bd0r[...],
            0.0,
        )
        pe = (
            jnp.dot(pe, wd1r[...], preferred_element_type=jnp.float32)
            + bd1r[...]
        ).reshape(BN, K, td)
        t = (qq[:, None, :] - kg + pe).reshape(BN * K, td)
        a = jnp.maximum(
            jnp.dot(t, wg0r[...], preferred_element_type=jnp.float32)
            + bg0r[...],
            0.0,
        )
        a = (
            jnp.dot(a, wg1r[...], preferred_element_type=jnp.float32)
            + bg1r[...]
        ).reshape(BN, K, td) * scale
        m = jnp.max(a, axis=1, keepdims=True)
        e = jnp.exp(a - m)
        attn = e / jnp.sum(e, axis=1, keepdims=True)
        res = jnp.sum(attn * (vg + pe), axis=1)
        out_ref[0] = (
            jnp.dot(res, w2r[...], preferred_element_type=jnp.float32)
            + b2r[...]
            + x_ref[0]
        )

    in_specs = [
        pl.BlockSpec((1, BN, td), lambda b, n: (b, n, 0)),
        pl.BlockSpec((1, BN, K, D2), lambda b, n: (b, n, 0, 0)),
        pl.BlockSpec((1, BN, _PD), lambda b, n: (b, n, 0)),
        pl.BlockSpec((1, BN, d), lambda b, n: (b, n, 0)),
    ] + [
        pl.BlockSpec(f.shape, lambda b, n: (0,) * f.ndim) for f in ws
    ]
    return pl.pallas_call(
        body,
        grid=(B, N // BN),
        in_specs=in_specs,
        out_specs=pl.BlockSpec((1, BN, d), lambda b, n: (b, n, 0)),
        out_shape=jax.ShapeDtypeStruct((B, N, d), jnp.float32),
    )(q, gath, pos_pad, x, *ws)


def _ptb(x, pos_pad, knn_idx, p):
    q, table = _ptb_pre(x, pos_pad, p)
    N = table.shape[1]
    if knn_idx.shape[2] == N:
        gath = jnp.broadcast_to(
            table[:, None, :, :], (table.shape[0], N, N, table.shape[2])
        )
    else:
        gath = _gather_rows(table, knn_idx)
    return _ptb_post(q, gath, pos_pad, x, p)


def _td_post(gath, new_xyz_pad, p, C):
    B, M, K, Dg = gath.shape
    w1 = _pad_w(p["mlp1"]["W"], C + _PD)
    b1 = _row(p["mlp1"]["b"])
    w2, b2 = p["mlp2"]["W"], _row(p["mlp2"]["b"])
    bh = w2.shape[1]

    def body(g_ref, nx_ref, w1r, b1r, w2r, b2r, out_ref):
        g = g_ref[0]
        norm = g[..., C:] - nx_ref[0][:, None, :]
        feat = jnp.concatenate([g[..., :C], norm], axis=-1).reshape(
            M * K, C + _PD
        )
        h = jnp.maximum(
            jnp.dot(feat, w1r[...], preferred_element_type=jnp.float32)
            + b1r[...],
            0.0,
        )
        h = jnp.maximum(
            jnp.dot(h, w2r[...], preferred_element_type=jnp.float32)
            + b2r[...],
            0.0,
        )
        out_ref[0] = jnp.max(h.reshape(M, K, bh), axis=1)

    in_specs = [
        pl.BlockSpec((1, M, K, Dg), lambda b: (b, 0, 0, 0)),
        pl.BlockSpec((1, M, _PD), lambda b: (b, 0, 0)),
    ] + [
        pl.BlockSpec(f.shape, lambda b: (0,) * f.ndim)
        for f in (w1, b1, w2, b2)
    ]
    return pl.pallas_call(
        body,
        grid=(B,),
        in_specs=in_specs,
        out_specs=pl.BlockSpec((1, M, bh), lambda b: (b, 0, 0)),
        out_shape=jax.ShapeDtypeStruct((B, M, bh), jnp.float32),
    )(gath, new_xyz_pad, w1, b1, w2, b2)


def _tu_post(gath_flat, dvals, pts2, fc2):
    B, M2, d2 = pts2.shape
    d = gath_flat.shape[2]
    w2, b2 = fc2["W"], _row(fc2["b"])

    def body(g_ref, dv_ref, p2_ref, w2r, b2r, out_ref):
        g = g_ref[0].reshape(M2, 3, d)
        d3 = jnp.maximum(dv_ref[0], 0.0)
        r = 1.0 / (d3 + 1e-8)
        w = r / jnp.sum(r, axis=-1, keepdims=True)
        f2 = jnp.maximum(
            jnp.dot(p2_ref[0], w2r[...], preferred_element_type=jnp.float32)
            + b2r[...],
            0.0,
        )
        out_ref[0] = jnp.sum(g * w[:, :, None], axis=1) + f2

    in_specs = [
        pl.BlockSpec((1, M2 * 3, d), lambda b: (b, 0, 0)),
        pl.BlockSpec((1, M2, 3), lambda b: (b, 0, 0)),
        pl.BlockSpec((1, M2, d2), lambda b: (b, 0, 0)),
        pl.BlockSpec(w2.shape, lambda b: (0, 0)),
        pl.BlockSpec(b2.shape, lambda b: (0, 0)),
    ]
    return pl.pallas_call(
        body,
        grid=(B,),
        in_specs=in_specs,
        out_specs=pl.BlockSpec((1, M2, d), lambda b: (b, 0, 0)),
        out_shape=jax.ShapeDtypeStruct((B, M2, d), jnp.float32),
    )(gath_flat, dvals, pts2, w2, b2)


_N_NEI = 16
_N_BLOCKS = 4
_DSR = 4


def kernel(x, cat_vec, params):
    B, NPTS, _ = x.shape
    pos = _pad_pos(x)

    def lyr(p, r):
        return (p["W"], p["b"], r)

    h = _mlp(x, [lyr(params["bb_fc"][0], True), lyr(params["bb_fc"][1], False)])

    knn_cache = {}

    def knn_self(pos_pad):
        key = id(pos_pad)
        if key not in knn_cache:
            n = pos_pad.shape[1]
            k = min(_N_NEI, n)
            if k == n:
                knn_cache[key] = jnp.broadcast_to(
                    jnp.arange(n, dtype=jnp.int32)[None, None, :], (B, n, n)
                )
            else:
                knn_cache[key] = _knn(pos_pad, pos_pad, k)[0]
        return knn_cache[key]

    h = _ptb(h, pos, knn_self(pos), params["bb_ptb"])
    hidden = [(pos, h)]
    for i in range(_N_BLOCKS):
        npoint = NPTS // _DSR ** (i + 1)
        new_pos = _fps(pos, npoint)
        k = min(_N_NEI, pos.shape[1])
        idx, _ = _knn(new_pos, pos, k)
        C = h.shape[2]
        table = jnp.concatenate([h, pos], axis=-1)
        gath = _gather_rows(table, idx)
        h = _td_post(gath, new_pos, params["td"][i], C)
        pos = new_pos
        h = _ptb(h, pos, knn_self(pos), params["bb_tf"][i])
        hidden.append((pos, h))

    h = _mlp(
        h,
        [
            lyr(params["seg_fc"][0], True),
            lyr(params["seg_fc"][1], True),
            lyr(params["seg_fc"][2], False),
        ],
    )
    h = _ptb(h, pos, knn_self(pos), params["seg_ptb"])

    for i in range(_N_BLOCKS):
        pos2, pts2 = hidden[-i - 2]
        tu = params["tu"][i]
        feats1 = _mlp(h, [lyr(tu["fc1"], True)])
        idx3, dv3 = _knn(pos2, pos, 3, want_vals=True)
        gath = _gather_rows(feats1, idx3)
        d = feats1.shape[2]
        M2 = pos2.shape[1]
        h = _tu_post(gath.reshape(B, M2 * 3, d), dv3, pts2, tu["fc2"])
        pos = pos2
        h = _ptb(h, pos, knn_self(pos), params["seg_tf"][i])

    o = jnp.concatenate([h, cat_vec], axis=-1)
    o = _mlp(
        o,
        [
            lyr(params["out"][0], True),
            lyr(params["out"][1], True),
            lyr(params["out"][2], False),
        ],
    )
    return o
